# baseline (speedup 1.0000x reference)
import functools

import jax
import jax.numpy as jnp
from jax import lax
from jax.experimental import pallas as pl
from jax.experimental.pallas import tpu as pltpu
from jax.experimental.pallas import tpu_sc as plsc

_LANES = 16
_CH = 100
_IPAD = 104
_NBUF = 4


@functools.lru_cache(maxsize=None)
def _build(B, L, D, V):
    info = plsc.get_sparse_core_info()
    NC, NS = info.num_cores, info.num_subcores
    NW = NC * NS
    assert B % NW == 0 and L == 2 * _CH and D % _LANES == 0
    n_chunks = (B // NW) * 2
    n_vec = D // _LANES

    mesh = plsc.VectorSubcoreMesh(core_axis_name="c", subcore_axis_name="s")

    @functools.partial(
        pl.kernel,
        out_type=jax.ShapeDtypeStruct((B * L, D), jnp.float32),
        mesh=mesh,
        compiler_params=pltpu.CompilerParams(use_tc_tiling_on_sc=False),
        scratch_types=[
            pltpu.VMEM((n_chunks, _IPAD), jnp.int32),
            pltpu.VMEM((L, D), jnp.float32),
            [pltpu.VMEM((_IPAD, D), jnp.float32)] * _NBUF,
            [pltpu.SemaphoreType.DMA] * _NBUF,
            [pltpu.SemaphoreType.DMA] * _NBUF,
        ],
    )
    def emb(idx_hbm, pe_hbm, table_hbm, out_hbm, idx_v, pe_v, bufs, gsems,
            ssems):
        wid = lax.axis_index("s") * NC + lax.axis_index("c")
        pltpu.sync_copy(pe_hbm, pe_v)
        pltpu.sync_copy(idx_hbm.at[pl.ds(wid * n_chunks, n_chunks)], idx_v)
        out0 = wid * n_chunks * _CH

        def gather(chunk, b):
            pltpu.async_copy(table_hbm.at[idx_v.at[chunk]], bufs[b],
                             gsems[b])

        gather(0, 0)
        gather(1, 1)

        def quad_body(j, carry):
            for p in range(_NBUF):
                it = _NBUF * j + p
                it2 = it + 2
                b2 = (p + 2) % _NBUF

                @pl.when(it2 < n_chunks)
                def _prefetch():
                    @pl.when(it2 >= _NBUF)
                    def _retire():
                        pltpu.make_async_copy(
                            bufs[b2].at[pl.ds(0, _CH)],
                            out_hbm.at[pl.ds(0, _CH)], ssems[b2]).wait()
                    gather(it2, b2)

                pltpu.make_async_copy(
                    table_hbm.at[idx_v.at[0]], bufs[p], gsems[p]).wait()

                pb = (it % 2) * _CH

                def add_row(r, c2):
                    for cc in range(n_vec):
                        sl = pl.ds(cc * _LANES, _LANES)
                        bufs[p][r, sl] = bufs[p][r, sl] + pe_v[pb + r, sl]
                    return c2

                lax.fori_loop(0, _CH, add_row, 0, unroll=2)
                pltpu.async_copy(
                    bufs[p].at[pl.ds(0, _CH)],
                    out_hbm.at[pl.ds(out0 + it * _CH, _CH)], ssems[p])
            return carry

        lax.fori_loop(0, n_chunks // _NBUF, quad_body, 0)
        for p in range(_NBUF):
            pltpu.make_async_copy(
                bufs[p].at[pl.ds(0, _CH)], out_hbm.at[pl.ds(0, _CH)],
                ssems[p]).wait()

    return emb


def kernel(x, table, pe):
    B, L = x.shape
    V, D = table.shape
    idx = x.reshape(-1, _CH).astype(jnp.int32)
    idx = jnp.pad(idx, ((0, 0), (0, _IPAD - _CH)))
    pe2 = pe[0, :L, :].astype(jnp.float32)
    out = _build(B, L, D, V)(idx, pe2, table)
    return out.reshape(B, L, D)


# speedup vs baseline: 2.0097x; 2.0097x over previous
import functools

import jax
import jax.numpy as jnp
from jax import lax
from jax.experimental import pallas as pl
from jax.experimental.pallas import tpu as pltpu
from jax.experimental.pallas import tpu_sc as plsc

_LANES = 16
_NBUF = 4


@functools.lru_cache(maxsize=None)
def _build(B, L, D, V):
    info = plsc.get_sparse_core_info()
    NC, NS = info.num_cores, info.num_subcores
    NW = NC * NS
    assert B % (NW * _NBUF) == 0 and L % 2 == 0 and D % _LANES == 0
    n_seq = B // NW
    half = L // 2
    n_vec = D // _LANES

    mesh = plsc.VectorSubcoreMesh(core_axis_name="c", subcore_axis_name="s")

    @functools.partial(
        pl.kernel,
        out_type=jax.ShapeDtypeStruct((B * L, D), jnp.float32),
        mesh=mesh,
        scratch_types=[
            [pltpu.VMEM((2, half), jnp.int32)] * _NBUF,
            pltpu.VMEM((L, D), jnp.float32),
            [pltpu.VMEM((L, D), jnp.float32)] * _NBUF,
            [pltpu.SemaphoreType.DMA] * _NBUF,
            [pltpu.SemaphoreType.DMA] * _NBUF,
        ],
    )
    def emb(idx_hbm, pe_hbm, table_hbm, out_hbm, idxs, pe_v, bufs, gsems,
            ssems):
        wid = lax.axis_index("s") * NC + lax.axis_index("c")
        pltpu.sync_copy(pe_hbm, pe_v)
        seq0 = wid * n_seq

        def fetch(it, b):
            pltpu.sync_copy(idx_hbm.at[pl.ds(2 * (seq0 + it), 2)], idxs[b])
            pltpu.async_copy(
                table_hbm.at[idxs[b].at[0]], bufs[b].at[pl.ds(0, half)],
                gsems[b])
            pltpu.async_copy(
                table_hbm.at[idxs[b].at[1]], bufs[b].at[pl.ds(half, half)],
                gsems[b])

        def wait_gathers(b):
            for h in range(2):
                pltpu.make_async_copy(
                    table_hbm.at[idxs[b].at[h]],
                    bufs[b].at[pl.ds(h * half, half)], gsems[b]).wait()

        def wait_scatter(b):
            pltpu.make_async_copy(
                bufs[b], out_hbm.at[pl.ds(0, L)], ssems[b]).wait()

        fetch(0, 0)
        fetch(1, 1)

        def quad_body(j, carry):
            for p in range(_NBUF):
                it = _NBUF * j + p
                it2 = it + 2
                b2 = (p + 2) % _NBUF

                @pl.when(it2 < n_seq)
                def _prefetch():
                    @pl.when(it2 >= _NBUF)
                    def _retire():
                        wait_scatter(b2)
                    fetch(it2, b2)

                wait_gathers(p)

                def add_row(r, c2):
                    for cc in range(n_vec):
                        sl = pl.ds(cc * _LANES, _LANES)
                        bufs[p][r, sl] = bufs[p][r, sl] + pe_v[r, sl]
                    return c2

                lax.fori_loop(0, L, add_row, 0, unroll=2)
                pltpu.async_copy(
                    bufs[p], out_hbm.at[pl.ds((seq0 + it) * L, L)], ssems[p])
            return carry

        lax.fori_loop(0, n_seq // _NBUF, quad_body, 0)
        for p in range(_NBUF):
            wait_scatter(p)

    return emb


def kernel(x, table, pe):
    B, L = x.shape
    V, D = table.shape
    idx = x.reshape(-1, L // 2).astype(jnp.int32)
    pe2 = pe[0, :L, :].astype(jnp.float32)
    out = _build(B, L, D, V)(idx, pe2, table)
    return out.reshape(B, L, D)


# speedup vs baseline: 6.5200x; 3.2443x over previous
import functools

import jax
import jax.numpy as jnp
from jax import lax
from jax.experimental import pallas as pl
from jax.experimental.pallas import tpu as pltpu
from jax.experimental.pallas import tpu_sc as plsc

_LANES = 16
_NBUF = 4


@functools.lru_cache(maxsize=None)
def _build(B, L, D, V):
    info = plsc.get_sparse_core_info()
    NC, NS = info.num_cores, info.num_subcores
    NW = NC * NS
    assert B % (NW * _NBUF) == 0 and L % 2 == 0 and D % _LANES == 0
    n_seq = B // NW
    half = L // 2

    mesh = plsc.VectorSubcoreMesh(core_axis_name="c", subcore_axis_name="s")

    @functools.partial(
        pl.kernel,
        out_type=jax.ShapeDtypeStruct((B * L, D), jnp.float32),
        mesh=mesh,
        scratch_types=[
            [pltpu.VMEM((2, half), jnp.int32)] * _NBUF,
            pltpu.VMEM_SHARED((L, D), jnp.float32),
            [pltpu.VMEM((L, D), jnp.float32)] * _NBUF,
            [pltpu.SemaphoreType.DMA] * _NBUF,
            [pltpu.SemaphoreType.DMA] * _NBUF,
            [pltpu.SemaphoreType.DMA] * _NBUF,
        ],
    )
    def emb(idx_hbm, pe_hbm, table_hbm, out_hbm, idxs, pe_sh, bufs, psems,
            gsems, ssems):
        wid = lax.axis_index("s") * NC + lax.axis_index("c")
        seq0 = wid * n_seq

        @pl.when(lax.axis_index("s") == 0)
        def _stage_pe():
            pltpu.sync_copy(pe_hbm, pe_sh)

        plsc.subcore_barrier()

        def stage(it, b):
            pltpu.async_copy(pe_sh, bufs[b], psems[b])
            pltpu.sync_copy(idx_hbm.at[pl.ds(2 * (seq0 + it), 2)], idxs[b])

        def gather_add(b):
            pltpu.make_async_copy(pe_sh, bufs[b], psems[b]).wait()
            for h in range(2):
                pltpu.async_copy(
                    table_hbm.at[idxs[b].at[h]],
                    bufs[b].at[pl.ds(h * half, half)], gsems[b], add=True)

        def wait_gathers(b):
            for h in range(2):
                pltpu.make_async_copy(
                    table_hbm.at[idxs[b].at[h]],
                    bufs[b].at[pl.ds(h * half, half)], gsems[b]).wait()

        def wait_scatter(b):
            pltpu.make_async_copy(
                bufs[b], out_hbm.at[pl.ds(0, L)], ssems[b]).wait()

        stage(0, 0)
        stage(1, 1)
        gather_add(0)

        def quad_body(j, carry):
            for p in range(_NBUF):
                it = _NBUF * j + p
                b1 = (p + 1) % _NBUF
                b2 = (p + 2) % _NBUF

                @pl.when(it + 2 < n_seq)
                def _prefetch():
                    @pl.when(it + 2 >= _NBUF)
                    def _retire():
                        wait_scatter(b2)
                    stage(it + 2, b2)

                @pl.when(it + 1 < n_seq)
                def _launch():
                    gather_add(b1)

                wait_gathers(p)
                pltpu.async_copy(
                    bufs[p], out_hbm.at[pl.ds((seq0 + it) * L, L)], ssems[p])
            return carry

        lax.fori_loop(0, n_seq // _NBUF, quad_body, 0)
        for p in range(_NBUF):
            wait_scatter(p)

    return emb


def kernel(x, table, pe):
    B, L = x.shape
    V, D = table.shape
    idx = x.reshape(-1, L // 2).astype(jnp.int32)
    pe2 = pe[0, :L, :].astype(jnp.float32)
    out = _build(B, L, D, V)(idx, pe2, table)
    return out.reshape(B, L, D)


# speedup vs baseline: 6.5523x; 1.0049x over previous
import functools

import jax
import jax.numpy as jnp
from jax import lax
from jax.experimental import pallas as pl
from jax.experimental.pallas import tpu as pltpu
from jax.experimental.pallas import tpu_sc as plsc

_LANES = 16
_NBUF = 4


@functools.lru_cache(maxsize=None)
def _build(B, L, D, V):
    info = plsc.get_sparse_core_info()
    NC, NS = info.num_cores, info.num_subcores
    NW = NC * NS
    assert B % (NW * _NBUF) == 0 and L % 2 == 0 and D % _LANES == 0
    n_seq = B // NW
    half = L // 2

    mesh = plsc.VectorSubcoreMesh(core_axis_name="c", subcore_axis_name="s")

    @functools.partial(
        pl.kernel,
        out_type=jax.ShapeDtypeStruct((B * L, D), jnp.float32),
        mesh=mesh,
        scratch_types=[
            [pltpu.VMEM((2, half), jnp.int32)] * _NBUF,
            pltpu.VMEM_SHARED((L, D), jnp.float32),
            [pltpu.VMEM((L, D), jnp.float32)] * _NBUF,
            [pltpu.SemaphoreType.DMA] * _NBUF,
            [pltpu.SemaphoreType.DMA] * _NBUF,
            [pltpu.SemaphoreType.DMA] * _NBUF,
            [pltpu.SemaphoreType.DMA] * _NBUF,
        ],
    )
    def emb(idx_hbm, pe_hbm, table_hbm, out_hbm, idxs, pe_sh, bufs, isems,
            psems, gsems, ssems):
        wid = lax.axis_index("s") * NC + lax.axis_index("c")
        seq0 = wid * n_seq

        @pl.when(lax.axis_index("s") == 0)
        def _stage_pe():
            pltpu.sync_copy(pe_hbm, pe_sh)

        plsc.subcore_barrier()

        def stage(it, b):
            pltpu.async_copy(pe_sh, bufs[b], psems[b])
            pltpu.async_copy(idx_hbm.at[pl.ds(2 * (seq0 + it), 2)], idxs[b],
                             isems[b])

        def gather_add(b):
            pltpu.make_async_copy(
                idx_hbm.at[pl.ds(0, 2)], idxs[b], isems[b]).wait()
            pltpu.make_async_copy(pe_sh, bufs[b], psems[b]).wait()
            for h in range(2):
                pltpu.async_copy(
                    table_hbm.at[idxs[b].at[h]],
                    bufs[b].at[pl.ds(h * half, half)], gsems[b], add=True)

        def wait_gathers(b):
            for h in range(2):
                pltpu.make_async_copy(
                    table_hbm.at[idxs[b].at[h]],
                    bufs[b].at[pl.ds(h * half, half)], gsems[b]).wait()

        def wait_scatter(b):
            pltpu.make_async_copy(
                bufs[b], out_hbm.at[pl.ds(0, L)], ssems[b]).wait()

        stage(0, 0)
        stage(1, 1)
        gather_add(0)

        def quad_body(j, carry):
            for p in range(_NBUF):
                it = _NBUF * j + p
                b1 = (p + 1) % _NBUF
                b2 = (p + 2) % _NBUF

                @pl.when(it + 2 < n_seq)
                def _prefetch():
                    @pl.when(it + 2 >= _NBUF)
                    def _retire():
                        wait_scatter(b2)
                    stage(it + 2, b2)

                @pl.when(it + 1 < n_seq)
                def _launch():
                    gather_add(b1)

                wait_gathers(p)
                pltpu.async_copy(
                    bufs[p], out_hbm.at[pl.ds((seq0 + it) * L, L)], ssems[p])
            return carry

        lax.fori_loop(0, n_seq // _NBUF, quad_body, 0)
        for p in range(_NBUF):
            wait_scatter(p)

    return emb


def kernel(x, table, pe):
    B, L = x.shape
    V, D = table.shape
    idx = x.reshape(-1, L // 2).astype(jnp.int32)
    pe2 = pe[0, :L, :].astype(jnp.float32)
    out = _build(B, L, D, V)(idx, pe2, table)
    return out.reshape(B, L, D)
